# speedup vs baseline: 1.0120x; 1.0120x over previous
import functools
import math

import jax
import jax.numpy as jnp
from jax import lax
from jax.experimental import pallas as pl
from jax.experimental.pallas import tpu as pltpu
from jax.experimental.pallas import tpu_sc as plsc

VOCAB = 100000
D_MODEL = 1024
SCALE = math.sqrt(D_MODEL)
LANES = 16
VECS_PER_ROW = D_MODEL // LANES

NUM_CORES = 2
NUM_SUBCORES = 16
NW = NUM_CORES * NUM_SUBCORES

B_TOTAL = 16384
B_PER_W = B_TOTAL // NW
CHUNK = 64
NCHUNK = B_PER_W // CHUNK


def _emb_body(idx_hbm, table_hbm, out_hbm, idx_v, rows_v, sem):
    wid = lax.axis_index("s") * NUM_CORES + lax.axis_index("c")
    base = wid * B_PER_W
    pltpu.sync_copy(idx_hbm.at[pl.ds(base, B_PER_W)], idx_v)

    for c in range(NCHUNK):
        pltpu.async_copy(
            table_hbm.at[idx_v.at[pl.ds(c * CHUNK, CHUNK)]], rows_v, sem
        ).wait()

        def scale_row(r, _):
            for v in range(VECS_PER_ROW):
                sl = pl.ds(v * LANES, LANES)
                rows_v[r, sl] = rows_v[r, sl] * SCALE
            return 0

        lax.fori_loop(0, CHUNK, scale_row, 0)

        pltpu.sync_copy(rows_v, out_hbm.at[pl.ds(base + c * CHUNK, CHUNK)])


@jax.jit
def _embed(x_flat, table):
    mesh = plsc.VectorSubcoreMesh(core_axis_name="c", subcore_axis_name="s")
    out = pl.kernel(
        _emb_body,
        out_type=jax.ShapeDtypeStruct((B_TOTAL, D_MODEL), jnp.float32),
        mesh=mesh,
        scratch_types=[
            pltpu.VMEM((B_PER_W,), jnp.int32),
            pltpu.VMEM((CHUNK, D_MODEL), jnp.float32),
            pltpu.SemaphoreType.DMA,
        ],
    )(x_flat, table)
    return out


def kernel(x, table):
    x_flat = x.reshape(-1).astype(jnp.int32)
    out = _embed(x_flat, table)
    return out.reshape(x.shape[0], x.shape[1], D_MODEL)


# speedup vs baseline: 1.4244x; 1.4075x over previous
import functools
import math

import jax
import jax.numpy as jnp
from jax import lax
from jax.experimental import pallas as pl
from jax.experimental.pallas import tpu as pltpu
from jax.experimental.pallas import tpu_sc as plsc

VOCAB = 100000
D_MODEL = 1024
SCALE = math.sqrt(D_MODEL)
LANES = 16
VECS_PER_ROW = D_MODEL // LANES

NUM_CORES = 2
NUM_SUBCORES = 16
NW = NUM_CORES * NUM_SUBCORES

B_TOTAL = 16384
B_PER_W = B_TOTAL // NW
CHUNK = 32
NCHUNK = B_PER_W // CHUNK
NBUF = 3


def _emb_body(idx_hbm, table_hbm, out_hbm, idx_v, rows_v, gsems, osems):
    wid = lax.axis_index("s") * NUM_CORES + lax.axis_index("c")
    base = wid * B_PER_W
    pltpu.sync_copy(idx_hbm.at[pl.ds(base, B_PER_W)], idx_v)

    gcopy = [None] * NBUF
    ocopy = [None] * NBUF

    def start_gather(n):
        b = n % NBUF
        if ocopy[b] is not None:
            ocopy[b].wait()
            ocopy[b] = None
        gcopy[b] = pltpu.async_copy(
            table_hbm.at[idx_v.at[pl.ds(n * CHUNK, CHUNK)]],
            rows_v.at[b],
            gsems.at[b],
        )

    for n in range(NBUF - 1):
        start_gather(n)

    for c in range(NCHUNK):
        b = c % NBUF
        n = c + NBUF - 1
        if n < NCHUNK:
            start_gather(n)

        gcopy[b].wait()

        def scale_row(r, _):
            for v in range(VECS_PER_ROW):
                sl = pl.ds(v * LANES, LANES)
                rows_v[b, r, sl] = rows_v[b, r, sl] * SCALE
            return 0

        lax.fori_loop(0, CHUNK, scale_row, 0)

        ocopy[b] = pltpu.async_copy(
            rows_v.at[b], out_hbm.at[pl.ds(base + c * CHUNK, CHUNK)], osems.at[b]
        )

    for b in range(NBUF):
        if ocopy[b] is not None:
            ocopy[b].wait()


@jax.jit
def _embed(x_flat, table):
    mesh = plsc.VectorSubcoreMesh(core_axis_name="c", subcore_axis_name="s")
    out = pl.kernel(
        _emb_body,
        out_type=jax.ShapeDtypeStruct((B_TOTAL, D_MODEL), jnp.float32),
        mesh=mesh,
        scratch_types=[
            pltpu.VMEM((B_PER_W,), jnp.int32),
            pltpu.VMEM((NBUF, CHUNK, D_MODEL), jnp.float32),
            pltpu.SemaphoreType.DMA((NBUF,)),
            pltpu.SemaphoreType.DMA((NBUF,)),
        ],
    )(x_flat, table)
    return out


def kernel(x, table):
    x_flat = x.reshape(-1).astype(jnp.int32)
    out = _embed(x_flat, table)
    return out.reshape(x.shape[0], x.shape[1], D_MODEL)
